# Initial kernel scaffold; baseline (speedup 1.0000x reference)
import jax
import jax.numpy as jnp
from jax.experimental import pallas as pl


def kernel(x, weight):
    raise NotImplementedError("write your pallas kernel here")


# speedup vs baseline: 3.3851x; 3.3851x over previous
import functools

import jax
import jax.numpy as jnp
from jax import lax
from jax.experimental import pallas as pl
from jax.experimental.pallas import tpu as pltpu
from jax.experimental.pallas import tpu_sc as plsc

D = 128
NC = 2
NS = 16
NW = NC * NS
CH = 128
K = 4


@functools.partial(jax.jit, static_argnums=())
def _sc_gather(x_r, weight):
    nw, nchunk, ch = x_r.shape
    b_total = nw * nchunk * ch
    mesh = plsc.VectorSubcoreMesh(core_axis_name="c", subcore_axis_name="s")

    @functools.partial(
        pl.kernel,
        mesh=mesh,
        out_type=jax.ShapeDtypeStruct((b_total, D), jnp.float32),
        scratch_types=[
            pltpu.VMEM((nchunk, ch), jnp.int32),
            pltpu.VMEM((K, ch, D), jnp.float32),
            pltpu.SemaphoreType.DMA,
            pltpu.SemaphoreType.DMA,
        ],
    )
    def k(x_hbm, w_hbm, out_hbm, idx_v, buf_v, gsem, ssem):
        wid = lax.axis_index("s") * NC + lax.axis_index("c")
        base = wid * (nchunk * ch)
        pltpu.sync_copy(x_hbm.at[wid], idx_v)

        def group(g, carry):
            c0 = g * K
            gathers = []
            for b in range(K):
                gathers.append(
                    pltpu.async_copy(w_hbm.at[idx_v.at[c0 + b]], buf_v.at[b], gsem)
                )
            for b in range(K):
                gathers[b].wait()
                row0 = base + (c0 + b) * ch
                pltpu.async_copy(buf_v.at[b], out_hbm.at[pl.ds(row0, ch)], ssem)
            for b in range(K):
                pltpu.make_async_copy(
                    buf_v.at[b], out_hbm.at[pl.ds(base + (c0 + b) * ch, ch)], ssem
                ).wait()
            return carry

        lax.fori_loop(0, nchunk // K, group, 0, unroll=False)

    return k(x_r, weight)


def kernel(x, weight):
    batch, fields = x.shape
    b_total = batch * fields
    assert b_total % (NW * CH * K) == 0
    nchunk = b_total // (NW * CH)
    x_r = x.reshape(NW, nchunk, CH).astype(jnp.int32)
    out = _sc_gather(x_r, weight)
    return out.reshape(batch, fields, D)
